# Initial kernel scaffold; baseline (speedup 1.0000x reference)
import jax
import jax.numpy as jnp
from jax.experimental import pallas as pl


def kernel(pred, targ):
    raise NotImplementedError("write your pallas kernel here")


# speedup vs baseline: 383.1894x; 383.1894x over previous
import functools

import jax
import jax.numpy as jnp
from jax import lax
from jax.experimental import pallas as pl
from jax.experimental.pallas import tpu as pltpu

_ZXY = (4.0, 64.0, 64.0)
_REAL = (25.0, 25.0, 4.0)
_CUTOFF = 2.0
_R, _C = 128, 128


def _acos_deg(x):
    xa = jnp.abs(x)
    s = jnp.sqrt(jnp.maximum(1.0 - xa, 0.0))
    p = ((-0.0187293 * xa + 0.0742610) * xa - 0.2121144) * xa + 1.5707288
    r = s * p
    r = jnp.where(x < 0.0, jnp.float32(jnp.pi) - r, r)
    return r * jnp.float32(180.0 / jnp.pi)


def _body(pred_ref, targ_ref, cm_ref, ms_ref):
    p = lambda ch: pred_ref[0, ch]
    t = lambda ch: targ_ref[0, ch]
    r = lax.broadcasted_iota(jnp.int32, (_R, _C), 0)
    c = lax.broadcasted_iota(jnp.int32, (_R, _C), 1)
    n = r * _C + c
    f0 = (n // 4096).astype(jnp.float32)
    f1 = ((n // 64) % 64).astype(jnp.float32)
    f2 = (n % 64).astype(jnp.float32)

    conf = jnp.float32(1.0) / (jnp.float32(1.0) + jnp.exp(-p(0)))
    valid = conf > jnp.float32(0.5)

    pos0 = (p(1) + f0) / _ZXY[0]
    pos1 = (p(2) + f1) / _ZXY[1]
    pos2 = (p(3) + f2) / _ZXY[2]
    pr0, pr1, pr2 = pos0 * _REAL[0], pos1 * _REAL[1], pos2 * _REAL[2]

    tg_valid = t(0) > jnp.float32(0.5)
    tq0 = (t(1) + f0) / _ZXY[0] * _REAL[0]
    tq1 = (t(2) + f1) / _ZXY[1] * _REAL[1]
    tq2 = (t(3) + f2) / _ZXY[2] * _REAL[2]

    bf = lambda x: x.astype(jnp.bfloat16).astype(jnp.float32)
    nh0, nh1, nh2 = bf(pos0), bf(pos1), bf(pos2)
    n2 = (pos0 * pos0 + pos1 * pos1) + pos2 * pos2
    ph0, ph1, ph2 = bf(pr0), bf(pr1), bf(pr2)
    p2r = (pr0 * pr0 + pr1 * pr1) + pr2 * pr2
    th0, th1, th2 = bf(tq0), bf(tq1), bf(tq2)
    t2r = (tq0 * tq0 + tq1 * tq1) + tq2 * tq2

    tx0, tx1, tx2 = t(4), t(5), t(6)
    ty0, ty1, ty2 = t(7), t(8), t(9)
    tz0 = tx1 * ty2 - tx2 * ty1
    tz1 = tx2 * ty0 - tx0 * ty2
    tz2 = tx0 * ty1 - tx1 * ty0
    ntx = jnp.sqrt(tx0 * tx0 + tx1 * tx1 + tx2 * tx2)
    nty = jnp.sqrt(ty0 * ty0 + ty1 * ty1 + ty2 * ty2)
    ntz = jnp.sqrt(tz0 * tz0 + tz1 * tz1 + tz2 * tz2)

    def cond(carry):
        return jnp.max(carry[0]) > jnp.float32(0.0)

    def body(carry):
        active_f, matched_f, npd, tp, wsum = carry
        active = active_f > jnp.float32(0.0)
        matched = matched_f > jnp.float32(0.0)
        confm = jnp.where(active, conf, jnp.float32(-1.0))
        m = jnp.max(confm)
        sidx = jnp.min(jnp.where(active & (conf == m), n, jnp.int32(1 << 30)))
        selm = (n == sidx).astype(jnp.float32)

        s0 = jnp.sum(nh0 * selm)
        s1 = jnp.sum(nh1 * selm)
        s2 = jnp.sum(nh2 * selm)
        s2n = jnp.sum(n2 * selm)
        ab = (nh0 * s0 + nh1 * s1) + nh2 * s2
        d2 = (n2 + s2n) - jnp.float32(2.0) * ab
        active = active & ~(d2 < jnp.float32(_CUTOFF * _CUTOFF))

        sr0 = jnp.sum(ph0 * selm)
        sr1 = jnp.sum(ph1 * selm)
        sr2 = jnp.sum(ph2 * selm)
        s2r = jnp.sum(p2r * selm)
        abt = (th0 * sr0 + th1 * sr1) + th2 * sr2
        dt2 = (t2r + s2r) - jnp.float32(2.0) * abt
        newly = tg_valid & ~matched & (dt2 < jnp.float32(_CUTOFF * _CUTOFF))

        ax0 = jnp.sum(p(4) * selm)
        ax1 = jnp.sum(p(5) * selm)
        ax2 = jnp.sum(p(6) * selm)
        ay0 = jnp.sum(p(7) * selm)
        ay1 = jnp.sum(p(8) * selm)
        ay2 = jnp.sum(p(9) * selm)
        az0 = ax1 * ay2 - ax2 * ay1
        az1 = ax2 * ay0 - ax0 * ay2
        az2 = ax0 * ay1 - ax1 * ay0
        nax = jnp.sqrt(ax0 * ax0 + ax1 * ax1 + ax2 * ax2)
        nay = jnp.sqrt(ay0 * ay0 + ay1 * ay1 + ay2 * ay2)
        naz = jnp.sqrt(az0 * az0 + az1 * az1 + az2 * az2)

        angsum = jnp.zeros((_R, _C), jnp.float32)
        for (a0, a1, a2, na, b0, b1, b2, nb) in (
            (ax0, ax1, ax2, nax, tx0, tx1, tx2, ntx),
            (ay0, ay1, ay2, nay, ty0, ty1, ty2, nty),
            (az0, az1, az2, naz, tz0, tz1, tz2, ntz),
        ):
            dot = b0 * a0 + b1 * a1 + b2 * a2
            cos = jnp.clip(dot / (na * nb), -1.0, 1.0)
            angsum = angsum + _acos_deg(cos)

        wsum = wsum + jnp.sum(jnp.where(newly, angsum, jnp.float32(0.0)))
        tp = tp + jnp.sum(newly.astype(jnp.int32))
        matched = matched | newly
        return (
            active.astype(jnp.float32),
            matched.astype(jnp.float32),
            npd + 1,
            tp,
            wsum,
        )

    init = (
        valid.astype(jnp.float32),
        jnp.zeros((_R, _C), jnp.float32),
        jnp.int32(0),
        jnp.int32(0),
        jnp.float32(0.0),
    )
    _, _, npd, tp, wsum = lax.while_loop(cond, body, init)

    ntg = jnp.sum(tg_valid.astype(jnp.int32))
    io3 = lax.broadcasted_iota(jnp.int32, (1, 1, 3), 2)
    cm_ref[...] = jnp.where(io3 == 0, tp, jnp.where(io3 == 1, npd - tp, ntg - tp))
    denom = jnp.maximum(jnp.float32(3) * tp.astype(jnp.float32), jnp.float32(1.0))
    ms = jnp.where(tp > 0, wsum / denom, jnp.float32(0.0))
    ms_ref[...] = jnp.broadcast_to(ms, (1, 1, 1))


@jax.jit
def kernel(pred, targ):
    B = pred.shape[0]
    pt = pred.reshape(B, _R * _C, 10).transpose(0, 2, 1).reshape(B, 10, _R, _C)
    tt = targ.reshape(B, _R * _C, 10).transpose(0, 2, 1).reshape(B, 10, _R, _C)
    cm, ms = pl.pallas_call(
        _body,
        grid=(B,),
        in_specs=[
            pl.BlockSpec((1, 10, _R, _C), lambda b: (b, 0, 0, 0)),
            pl.BlockSpec((1, 10, _R, _C), lambda b: (b, 0, 0, 0)),
        ],
        out_specs=[
            pl.BlockSpec((1, 1, 3), lambda b: (b, 0, 0)),
            pl.BlockSpec((1, 1, 1), lambda b: (b, 0, 0)),
        ],
        out_shape=[
            jax.ShapeDtypeStruct((B, 1, 3), jnp.int32),
            jax.ShapeDtypeStruct((B, 1, 1), jnp.float32),
        ],
    )(pt, tt)
    return cm.reshape(B, 1, 1, 3), ms.reshape(B, 1, 1, 1)


# speedup vs baseline: 391.1422x; 1.0208x over previous
import jax
import jax.numpy as jnp
from jax import lax
from jax.experimental import pallas as pl

_ZXY = (4.0, 64.0, 64.0)
_REAL = (25.0, 25.0, 4.0)
_CUT2 = 4.0
_B = 4
_R, _C = 128, 128


def _acos_deg(x):
    xa = jnp.abs(x)
    s = jnp.sqrt(jnp.maximum(1.0 - xa, 0.0))
    p = ((-0.0187293 * xa + 0.0742610) * xa - 0.2121144) * xa + 1.5707288
    r = s * p
    r = jnp.where(x < 0.0, jnp.float32(jnp.pi) - r, r)
    return r * jnp.float32(180.0 / jnp.pi)


def _bsum(x):
    return jnp.sum(x, axis=(1, 2), keepdims=True)


def _bmax(x):
    return jnp.max(x, axis=(1, 2), keepdims=True)


def _bmin(x):
    return jnp.min(x, axis=(1, 2), keepdims=True)


def _body(pred_ref, targ_ref, cm_ref, ms_ref):
    p = lambda ch: pred_ref[:, ch]
    t = lambda ch: targ_ref[:, ch]
    r = lax.broadcasted_iota(jnp.int32, (_B, _R, _C), 1)
    c = lax.broadcasted_iota(jnp.int32, (_B, _R, _C), 2)
    n = r * _C + c
    f0 = (n // 4096).astype(jnp.float32)
    f1 = ((n // 64) % 64).astype(jnp.float32)
    f2 = (n % 64).astype(jnp.float32)

    conf = jnp.float32(1.0) / (jnp.float32(1.0) + jnp.exp(-p(0)))
    valid = conf > jnp.float32(0.5)

    pos0 = (p(1) + f0) / _ZXY[0]
    pos1 = (p(2) + f1) / _ZXY[1]
    pos2 = (p(3) + f2) / _ZXY[2]
    pr0, pr1, pr2 = pos0 * _REAL[0], pos1 * _REAL[1], pos2 * _REAL[2]

    tg_valid = t(0) > jnp.float32(0.5)
    tq0 = (t(1) + f0) / _ZXY[0] * _REAL[0]
    tq1 = (t(2) + f1) / _ZXY[1] * _REAL[1]
    tq2 = (t(3) + f2) / _ZXY[2] * _REAL[2]

    bf = lambda x: x.astype(jnp.bfloat16).astype(jnp.float32)
    nh0, nh1, nh2 = bf(pos0), bf(pos1), bf(pos2)
    n2 = (pos0 * pos0 + pos1 * pos1) + pos2 * pos2
    ph0, ph1, ph2 = bf(pr0), bf(pr1), bf(pr2)
    p2r = (pr0 * pr0 + pr1 * pr1) + pr2 * pr2
    th0, th1, th2 = bf(tq0), bf(tq1), bf(tq2)
    t2r = (tq0 * tq0 + tq1 * tq1) + tq2 * tq2

    tx0, tx1, tx2 = t(4), t(5), t(6)
    ty0, ty1, ty2 = t(7), t(8), t(9)
    tz0 = tx1 * ty2 - tx2 * ty1
    tz1 = tx2 * ty0 - tx0 * ty2
    tz2 = tx0 * ty1 - tx1 * ty0
    ntx = jnp.sqrt(tx0 * tx0 + tx1 * tx1 + tx2 * tx2)
    nty = jnp.sqrt(ty0 * ty0 + ty1 * ty1 + ty2 * ty2)
    ntz = jnp.sqrt(tz0 * tz0 + tz1 * tz1 + tz2 * tz2)

    def cond(carry):
        return jnp.max(carry[0]) > jnp.float32(0.0)

    def body(carry):
        active_f, matched_f, npd, tp, wsum = carry
        active = active_f > jnp.float32(0.0)
        matched = matched_f > jnp.float32(0.0)
        confm = jnp.where(active, conf, jnp.float32(-1.0))
        m = _bmax(confm)
        live = m > jnp.float32(0.0)
        sidx = _bmin(jnp.where(active & (conf == m), n, jnp.int32(1 << 30)))
        selm = (n == sidx).astype(jnp.float32)

        s0 = _bsum(nh0 * selm)
        s1 = _bsum(nh1 * selm)
        s2 = _bsum(nh2 * selm)
        s2n = _bsum(n2 * selm)
        ab = (nh0 * s0 + nh1 * s1) + nh2 * s2
        d2 = (n2 + s2n) - jnp.float32(2.0) * ab
        active = active & ~(d2 < jnp.float32(_CUT2))

        sr0 = _bsum(ph0 * selm)
        sr1 = _bsum(ph1 * selm)
        sr2 = _bsum(ph2 * selm)
        s2r = _bsum(p2r * selm)
        abt = (th0 * sr0 + th1 * sr1) + th2 * sr2
        dt2 = (t2r + s2r) - jnp.float32(2.0) * abt
        newly = tg_valid & ~matched & (dt2 < jnp.float32(_CUT2)) & live

        ax0 = _bsum(p(4) * selm)
        ax1 = _bsum(p(5) * selm)
        ax2 = _bsum(p(6) * selm)
        ay0 = _bsum(p(7) * selm)
        ay1 = _bsum(p(8) * selm)
        ay2 = _bsum(p(9) * selm)
        az0 = ax1 * ay2 - ax2 * ay1
        az1 = ax2 * ay0 - ax0 * ay2
        az2 = ax0 * ay1 - ax1 * ay0
        nax = jnp.sqrt(ax0 * ax0 + ax1 * ax1 + ax2 * ax2)
        nay = jnp.sqrt(ay0 * ay0 + ay1 * ay1 + ay2 * ay2)
        naz = jnp.sqrt(az0 * az0 + az1 * az1 + az2 * az2)

        angsum = jnp.zeros((_B, _R, _C), jnp.float32)
        for (a0, a1, a2, na, b0, b1, b2, nb) in (
            (ax0, ax1, ax2, nax, tx0, tx1, tx2, ntx),
            (ay0, ay1, ay2, nay, ty0, ty1, ty2, nty),
            (az0, az1, az2, naz, tz0, tz1, tz2, ntz),
        ):
            dot = b0 * a0 + b1 * a1 + b2 * a2
            cos = jnp.clip(dot / (na * nb), -1.0, 1.0)
            angsum = angsum + _acos_deg(cos)

        newly_f = newly.astype(jnp.float32)
        wsum = wsum + _bsum(angsum * newly_f)[:, 0, 0]
        tp = tp + _bsum(newly_f)[:, 0, 0].astype(jnp.int32)
        npd = npd + live[:, 0, 0].astype(jnp.int32)
        matched = matched | newly
        return (
            active.astype(jnp.float32),
            matched.astype(jnp.float32),
            npd,
            tp,
            wsum,
        )

    init = (
        valid.astype(jnp.float32),
        jnp.zeros((_B, _R, _C), jnp.float32),
        jnp.zeros((_B,), jnp.int32),
        jnp.zeros((_B,), jnp.int32),
        jnp.zeros((_B,), jnp.float32),
    )
    _, _, npd, tp, wsum = lax.while_loop(cond, body, init)

    ntg = _bsum(tg_valid.astype(jnp.float32))[:, 0, 0].astype(jnp.int32)
    io3 = lax.broadcasted_iota(jnp.int32, (_B, 1, 3), 2)
    tp3 = tp[:, None, None]
    cm_ref[...] = jnp.where(
        io3 == 0, tp3, jnp.where(io3 == 1, (npd - tp)[:, None, None], (ntg - tp)[:, None, None])
    )
    tpf = tp.astype(jnp.float32)
    denom = jnp.maximum(jnp.float32(3) * tpf, jnp.float32(1.0))
    ms = jnp.where(tp > 0, wsum / denom, jnp.float32(0.0))
    ms_ref[...] = ms[:, None, None]


@jax.jit
def kernel(pred, targ):
    B = pred.shape[0]
    pt = pred.reshape(B, _R * _C, 10).transpose(0, 2, 1).reshape(B, 10, _R, _C)
    tt = targ.reshape(B, _R * _C, 10).transpose(0, 2, 1).reshape(B, 10, _R, _C)
    cm, ms = pl.pallas_call(
        _body,
        out_shape=[
            jax.ShapeDtypeStruct((B, 1, 3), jnp.int32),
            jax.ShapeDtypeStruct((B, 1, 1), jnp.float32),
        ],
    )(pt, tt)
    return cm.reshape(B, 1, 1, 3), ms.reshape(B, 1, 1, 1)
